# baseline (speedup 1.0000x reference)
import functools

import jax
import jax.numpy as jnp
from jax import lax
from jax.experimental import pallas as pl
from jax.experimental.pallas import tpu as pltpu
from jax.experimental.pallas import tpu_sc as plsc

N_NODES = 10000
N_EDGES = 320000
D_FEAT = 128
HIDDEN = 64
OUT = 32

NC = 2
NS = 16
NW = NC * NS
K = 80
NCHUNK = 125
PER_W = K * NCHUNK
N_PAD = 10240
ROWS_PER_TILE = N_PAD // NS

_SC_MESH = plsc.VectorSubcoreMesh(core_axis_name="c", subcore_axis_name="s")
_SC_PARAMS = pltpu.CompilerParams(use_tc_tiling_on_sc=False)


def _deg_body(dst_hbm, ones_hbm, zeros_hbm, out_hbm, didx_v, ones_v, acc):
    c = lax.axis_index("c")
    s = lax.axis_index("s")
    wid = c * NS + s
    pltpu.sync_copy(dst_hbm.at[wid], didx_v)
    pltpu.sync_copy(ones_hbm, ones_v)
    row0 = s * ROWS_PER_TILE
    pltpu.sync_copy(zeros_hbm.at[pl.ds(row0, ROWS_PER_TILE)],
                    acc.at[pl.ds(row0, ROWS_PER_TILE)])
    plsc.subcore_barrier()

    def body(j, carry):
        pltpu.sync_copy(ones_v, acc.at[didx_v.at[j]], add=True)
        return carry

    lax.fori_loop(0, NCHUNK, body, 0)
    plsc.subcore_barrier()
    pltpu.sync_copy(acc.at[pl.ds(row0, ROWS_PER_TILE)],
                    out_hbm.at[c, pl.ds(row0, ROWS_PER_TILE)])


_deg_kernel = functools.partial(
    pl.kernel,
    out_type=jax.ShapeDtypeStruct((NC, N_PAD, 16), jnp.float32),
    mesh=_SC_MESH,
    compiler_params=_SC_PARAMS,
    scratch_types=[
        pltpu.VMEM((NCHUNK, K), jnp.int32),
        pltpu.VMEM((K, 16), jnp.float32),
        pltpu.VMEM_SHARED((N_PAD, 16), jnp.float32),
    ],
)(_deg_body)


def _agg_body(h_hbm, src_hbm, dst_hbm, zeros_hbm, out_hbm,
              sidx_v, didx_v, rows0, rows1, rows2, sem0, sem1, sem2, acc):
    c = lax.axis_index("c")
    s = lax.axis_index("s")
    wid = c * NS + s
    pltpu.sync_copy(src_hbm.at[wid], sidx_v)
    pltpu.sync_copy(dst_hbm.at[wid], didx_v)
    base = s * ROWS_PER_TILE
    pltpu.sync_copy(zeros_hbm.at[pl.ds(base, ROWS_PER_TILE)],
                    acc.at[pl.ds(base, ROWS_PER_TILE)])
    plsc.subcore_barrier()

    bufs = (rows0, rows1, rows2)
    sems = (sem0, sem1, sem2)
    pltpu.async_copy(h_hbm.at[sidx_v.at[0]], rows0, sem0)
    pltpu.async_copy(h_hbm.at[sidx_v.at[1]], rows1, sem1)

    def body(i, carry):
        j = 3 * i
        for b in range(3):
            pltpu.async_copy(h_hbm.at[sidx_v.at[j + b + 2]],
                             bufs[(b + 2) % 3], sems[(b + 2) % 3])
            pltpu.make_async_copy(h_hbm.at[sidx_v.at[j + b]],
                                  bufs[b], sems[b]).wait()
            pltpu.sync_copy(bufs[b], acc.at[didx_v.at[j + b]], add=True)
        return carry

    lax.fori_loop(0, (NCHUNK - 2) // 3, body, 0)
    pltpu.make_async_copy(h_hbm.at[sidx_v.at[NCHUNK - 2]], rows0, sem0).wait()
    pltpu.sync_copy(rows0, acc.at[didx_v.at[NCHUNK - 2]], add=True)
    pltpu.make_async_copy(h_hbm.at[sidx_v.at[NCHUNK - 1]], rows1, sem1).wait()
    pltpu.sync_copy(rows1, acc.at[didx_v.at[NCHUNK - 1]], add=True)

    plsc.subcore_barrier()
    pltpu.sync_copy(acc.at[pl.ds(base, ROWS_PER_TILE)],
                    out_hbm.at[c, pl.ds(base, ROWS_PER_TILE)])


_agg_kernel = functools.partial(
    pl.kernel,
    out_type=jax.ShapeDtypeStruct((NC, N_PAD, HIDDEN), jnp.float32),
    mesh=_SC_MESH,
    compiler_params=_SC_PARAMS,
    scratch_types=[
        pltpu.VMEM((NCHUNK, K), jnp.int32),
        pltpu.VMEM((NCHUNK, K), jnp.int32),
        pltpu.VMEM((K, HIDDEN), jnp.float32),
        pltpu.VMEM((K, HIDDEN), jnp.float32),
        pltpu.VMEM((K, HIDDEN), jnp.float32),
        pltpu.SemaphoreType.DMA,
        pltpu.SemaphoreType.DMA,
        pltpu.SemaphoreType.DMA,
        pltpu.VMEM_SHARED((N_PAD, HIDDEN), jnp.float32),
    ],
)(_agg_body)


_BLK = 1000


def _mm_body(x_ref, w_ref, out_ref):
    out_ref[...] = jnp.dot(x_ref[...], w_ref[...],
                           preferred_element_type=jnp.float32)


def _tc_mm(x, W):
    return pl.pallas_call(
        _mm_body,
        grid=(N_NODES // _BLK,),
        in_specs=[
            pl.BlockSpec((_BLK, D_FEAT), lambda i: (i, 0)),
            pl.BlockSpec((D_FEAT, HIDDEN), lambda i: (0, 0)),
        ],
        out_specs=pl.BlockSpec((_BLK, HIDDEN), lambda i: (i, 0)),
        out_shape=jax.ShapeDtypeStruct((N_NODES, HIDDEN), jnp.float32),
    )(x, W)


def _dinv_of(degp_ref):
    deg = degp_ref[0, :, :1] + degp_ref[1, :, :1] + 1.0
    return lax.rsqrt(deg)


def _tc1_body(degp_ref, xw_ref, h1p_ref):
    h1p_ref[...] = xw_ref[...] * _dinv_of(degp_ref)


def _tc1(degp, xw):
    return pl.pallas_call(
        _tc1_body,
        grid=(N_NODES // _BLK,),
        in_specs=[
            pl.BlockSpec((NC, _BLK, 16), lambda i: (0, i, 0)),
            pl.BlockSpec((_BLK, HIDDEN), lambda i: (i, 0)),
        ],
        out_specs=pl.BlockSpec((_BLK, HIDDEN), lambda i: (i, 0)),
        out_shape=jax.ShapeDtypeStruct((N_NODES, HIDDEN), jnp.float32),
    )(degp, xw)


def _tc2_body(degp_ref, S_ref, hp_ref, b_ref, w_ref, out_ref):
    dinv = _dinv_of(degp_ref)
    h = (S_ref[0] + S_ref[1] + hp_ref[...]) * dinv + b_ref[...]
    h = jnp.maximum(h, 0.0)
    out_ref[...] = jnp.dot(h, w_ref[...],
                           preferred_element_type=jnp.float32) * dinv


def _tc2(degp, S, hp, b, W):
    return pl.pallas_call(
        _tc2_body,
        grid=(N_NODES // _BLK,),
        in_specs=[
            pl.BlockSpec((NC, _BLK, 16), lambda i: (0, i, 0)),
            pl.BlockSpec((NC, _BLK, HIDDEN), lambda i: (0, i, 0)),
            pl.BlockSpec((_BLK, HIDDEN), lambda i: (i, 0)),
            pl.BlockSpec((1, HIDDEN), lambda i: (0, 0)),
            pl.BlockSpec((HIDDEN, HIDDEN), lambda i: (0, 0)),
        ],
        out_specs=pl.BlockSpec((_BLK, HIDDEN), lambda i: (i, 0)),
        out_shape=jax.ShapeDtypeStruct((N_NODES, HIDDEN), jnp.float32),
    )(degp, S, hp, b, W)


def _tc3_body(degp_ref, S_ref, hp_ref, b_ref, wf_ref, bf_ref, out_ref):
    dinv = _dinv_of(degp_ref)
    h = (S_ref[0] + S_ref[1] + hp_ref[...]) * dinv + b_ref[...]
    h = jnp.maximum(h, 0.0)
    out_ref[...] = jnp.dot(h, wf_ref[...],
                           preferred_element_type=jnp.float32) + bf_ref[...]


def _tc3(degp, S, hp, b, Wf, bf):
    return pl.pallas_call(
        _tc3_body,
        grid=(N_NODES // _BLK,),
        in_specs=[
            pl.BlockSpec((NC, _BLK, 16), lambda i: (0, i, 0)),
            pl.BlockSpec((NC, _BLK, HIDDEN), lambda i: (0, i, 0)),
            pl.BlockSpec((_BLK, HIDDEN), lambda i: (i, 0)),
            pl.BlockSpec((1, HIDDEN), lambda i: (0, 0)),
            pl.BlockSpec((HIDDEN, OUT), lambda i: (0, 0)),
            pl.BlockSpec((1, OUT), lambda i: (0, 0)),
        ],
        out_specs=pl.BlockSpec((_BLK, OUT), lambda i: (i, 0)),
        out_shape=jax.ShapeDtypeStruct((N_NODES, OUT), jnp.float32),
    )(degp, S, hp, b, Wf, bf)


def kernel(x, edge_index, edge_attr, W1, b1, W2, b2, Wf, bf):
    del edge_attr
    src = edge_index[0].astype(jnp.int32).reshape(NW, NCHUNK, K)
    dst = edge_index[1].astype(jnp.int32).reshape(NW, NCHUNK, K)
    ones16 = jnp.ones((K, 16), jnp.float32)
    zeros16 = jnp.zeros((N_PAD, 16), jnp.float32)
    zeros64 = jnp.zeros((N_PAD, HIDDEN), jnp.float32)

    degp = _deg_kernel(dst, ones16, zeros16)
    xw = _tc_mm(x, W1)
    h1p = _tc1(degp, xw)
    S1 = _agg_kernel(h1p, src, dst, zeros64)
    h2p = _tc2(degp, S1, h1p, b1.reshape(1, HIDDEN), W2)
    S2 = _agg_kernel(h2p, src, dst, zeros64)
    return _tc3(degp, S2, h2p, b2.reshape(1, HIDDEN), Wf, bf.reshape(1, OUT))


# speedup vs baseline: 1.0714x; 1.0714x over previous
import functools

import jax
import jax.numpy as jnp
from jax import lax
from jax.experimental import pallas as pl
from jax.experimental.pallas import tpu as pltpu
from jax.experimental.pallas import tpu_sc as plsc

N_NODES = 10000
N_EDGES = 320000
D_FEAT = 128
HIDDEN = 64
OUT = 32

NC = 2
NS = 16
NW = NC * NS
K = 128
NCHUNK = 80
PER_W = K * NCHUNK
E_PAD = NW * PER_W
N_PAD = 10240
ROWS_PER_TILE = N_PAD // NS

_SC_MESH = plsc.VectorSubcoreMesh(core_axis_name="c", subcore_axis_name="s")
_SC_PARAMS = pltpu.CompilerParams(use_tc_tiling_on_sc=False)


def _deg_body(dst_hbm, ones_hbm, zeros_hbm, out_hbm, didx_v, ones_v, acc):
    c = lax.axis_index("c")
    s = lax.axis_index("s")
    wid = c * NS + s
    pltpu.sync_copy(dst_hbm.at[wid], didx_v)
    pltpu.sync_copy(ones_hbm, ones_v)
    row0 = s * ROWS_PER_TILE
    pltpu.sync_copy(zeros_hbm.at[pl.ds(row0, ROWS_PER_TILE)],
                    acc.at[pl.ds(row0, ROWS_PER_TILE)])
    plsc.subcore_barrier()

    def body(j, carry):
        pltpu.sync_copy(ones_v, acc.at[didx_v.at[j]], add=True)
        return carry

    lax.fori_loop(0, NCHUNK, body, 0)
    plsc.subcore_barrier()
    pltpu.sync_copy(acc.at[pl.ds(row0, ROWS_PER_TILE)],
                    out_hbm.at[c, pl.ds(row0, ROWS_PER_TILE)])


_deg_kernel = functools.partial(
    pl.kernel,
    out_type=jax.ShapeDtypeStruct((NC, N_PAD, 16), jnp.float32),
    mesh=_SC_MESH,
    compiler_params=_SC_PARAMS,
    scratch_types=[
        pltpu.VMEM((NCHUNK, K), jnp.int32),
        pltpu.VMEM((K, 16), jnp.float32),
        pltpu.VMEM_SHARED((N_PAD, 16), jnp.float32),
    ],
)(_deg_body)


def _agg_body(h_hbm, src_hbm, dst_hbm, zeros_hbm, out_hbm,
              sidx_v, didx_v, rows0, rows1, rows2, sem0, sem1, sem2, acc):
    c = lax.axis_index("c")
    s = lax.axis_index("s")
    wid = c * NS + s
    pltpu.sync_copy(src_hbm.at[wid], sidx_v)
    pltpu.sync_copy(dst_hbm.at[wid], didx_v)
    base = s * ROWS_PER_TILE
    pltpu.sync_copy(zeros_hbm.at[pl.ds(base, ROWS_PER_TILE)],
                    acc.at[pl.ds(base, ROWS_PER_TILE)])
    plsc.subcore_barrier()

    bufs = (rows0, rows1, rows2)
    sems = (sem0, sem1, sem2)
    pltpu.async_copy(h_hbm.at[sidx_v.at[0]], rows0, sem0)
    pltpu.async_copy(h_hbm.at[sidx_v.at[1]], rows1, sem1)

    def body(i, carry):
        j = 3 * i
        for b in range(3):
            pltpu.async_copy(h_hbm.at[sidx_v.at[j + b + 2]],
                             bufs[(b + 2) % 3], sems[(b + 2) % 3])
            pltpu.make_async_copy(h_hbm.at[sidx_v.at[j + b]],
                                  bufs[b], sems[b]).wait()
            pltpu.sync_copy(bufs[b], acc.at[didx_v.at[j + b]], add=True)
        return carry

    lax.fori_loop(0, (NCHUNK - 2) // 3, body, 0)
    pltpu.make_async_copy(h_hbm.at[sidx_v.at[NCHUNK - 2]], rows0, sem0).wait()
    pltpu.sync_copy(rows0, acc.at[didx_v.at[NCHUNK - 2]], add=True)
    pltpu.make_async_copy(h_hbm.at[sidx_v.at[NCHUNK - 1]], rows1, sem1).wait()
    pltpu.sync_copy(rows1, acc.at[didx_v.at[NCHUNK - 1]], add=True)

    plsc.subcore_barrier()
    pltpu.sync_copy(acc.at[pl.ds(base, ROWS_PER_TILE)],
                    out_hbm.at[c, pl.ds(base, ROWS_PER_TILE)])


_agg_kernel = functools.partial(
    pl.kernel,
    out_type=jax.ShapeDtypeStruct((NC, N_PAD, HIDDEN), jnp.float32),
    mesh=_SC_MESH,
    compiler_params=_SC_PARAMS,
    scratch_types=[
        pltpu.VMEM((NCHUNK, K), jnp.int32),
        pltpu.VMEM((NCHUNK, K), jnp.int32),
        pltpu.VMEM((K, HIDDEN), jnp.float32),
        pltpu.VMEM((K, HIDDEN), jnp.float32),
        pltpu.VMEM((K, HIDDEN), jnp.float32),
        pltpu.SemaphoreType.DMA,
        pltpu.SemaphoreType.DMA,
        pltpu.SemaphoreType.DMA,
        pltpu.VMEM_SHARED((N_PAD, HIDDEN), jnp.float32),
    ],
)(_agg_body)


_BLK = 1000


def _mm_body(x_ref, w_ref, out_ref):
    out_ref[...] = jnp.dot(x_ref[...], w_ref[...],
                           preferred_element_type=jnp.float32)


def _tc_mm(x, W):
    return pl.pallas_call(
        _mm_body,
        grid=(N_NODES // _BLK,),
        in_specs=[
            pl.BlockSpec((_BLK, D_FEAT), lambda i: (i, 0)),
            pl.BlockSpec((D_FEAT, HIDDEN), lambda i: (0, 0)),
        ],
        out_specs=pl.BlockSpec((_BLK, HIDDEN), lambda i: (i, 0)),
        out_shape=jax.ShapeDtypeStruct((N_NODES, HIDDEN), jnp.float32),
    )(x, W)


def _dinv_of(degp_ref):
    deg = degp_ref[0, :, :1] + degp_ref[1, :, :1] + 1.0
    return lax.rsqrt(deg)


def _tc1_body(degp_ref, xw_ref, h1p_ref):
    h1p_ref[...] = xw_ref[...] * _dinv_of(degp_ref)


def _tc1(degp, xw):
    return pl.pallas_call(
        _tc1_body,
        grid=(N_NODES // _BLK,),
        in_specs=[
            pl.BlockSpec((NC, _BLK, 16), lambda i: (0, i, 0)),
            pl.BlockSpec((_BLK, HIDDEN), lambda i: (i, 0)),
        ],
        out_specs=pl.BlockSpec((_BLK, HIDDEN), lambda i: (i, 0)),
        out_shape=jax.ShapeDtypeStruct((N_NODES, HIDDEN), jnp.float32),
    )(degp, xw)


def _tc2_body(degp_ref, S_ref, hp_ref, b_ref, w_ref, out_ref):
    dinv = _dinv_of(degp_ref)
    h = (S_ref[0] + S_ref[1] + hp_ref[...]) * dinv + b_ref[...]
    h = jnp.maximum(h, 0.0)
    out_ref[...] = jnp.dot(h, w_ref[...],
                           preferred_element_type=jnp.float32) * dinv


def _tc2(degp, S, hp, b, W):
    return pl.pallas_call(
        _tc2_body,
        grid=(N_NODES // _BLK,),
        in_specs=[
            pl.BlockSpec((NC, _BLK, 16), lambda i: (0, i, 0)),
            pl.BlockSpec((NC, _BLK, HIDDEN), lambda i: (0, i, 0)),
            pl.BlockSpec((_BLK, HIDDEN), lambda i: (i, 0)),
            pl.BlockSpec((1, HIDDEN), lambda i: (0, 0)),
            pl.BlockSpec((HIDDEN, HIDDEN), lambda i: (0, 0)),
        ],
        out_specs=pl.BlockSpec((_BLK, HIDDEN), lambda i: (i, 0)),
        out_shape=jax.ShapeDtypeStruct((N_NODES, HIDDEN), jnp.float32),
    )(degp, S, hp, b, W)


def _tc3_body(degp_ref, S_ref, hp_ref, b_ref, wf_ref, bf_ref, out_ref):
    dinv = _dinv_of(degp_ref)
    h = (S_ref[0] + S_ref[1] + hp_ref[...]) * dinv + b_ref[...]
    h = jnp.maximum(h, 0.0)
    out_ref[...] = jnp.dot(h, wf_ref[...],
                           preferred_element_type=jnp.float32) + bf_ref[...]


def _tc3(degp, S, hp, b, Wf, bf):
    return pl.pallas_call(
        _tc3_body,
        grid=(N_NODES // _BLK,),
        in_specs=[
            pl.BlockSpec((NC, _BLK, 16), lambda i: (0, i, 0)),
            pl.BlockSpec((NC, _BLK, HIDDEN), lambda i: (0, i, 0)),
            pl.BlockSpec((_BLK, HIDDEN), lambda i: (i, 0)),
            pl.BlockSpec((1, HIDDEN), lambda i: (0, 0)),
            pl.BlockSpec((HIDDEN, OUT), lambda i: (0, 0)),
            pl.BlockSpec((1, OUT), lambda i: (0, 0)),
        ],
        out_specs=pl.BlockSpec((_BLK, OUT), lambda i: (i, 0)),
        out_shape=jax.ShapeDtypeStruct((N_NODES, OUT), jnp.float32),
    )(degp, S, hp, b, Wf, bf)


def kernel(x, edge_index, edge_attr, W1, b1, W2, b2, Wf, bf):
    del edge_attr
    pad = jnp.arange(E_PAD - N_EDGES, dtype=jnp.int32)
    pad_src = (pad * 131) % N_NODES
    pad_dst = N_NODES + pad % (N_PAD - N_NODES)
    dst = jnp.concatenate([edge_index[1].astype(jnp.int32), pad_dst])
    dst = dst.reshape(NW, NCHUNK, K)
    src = jnp.concatenate([edge_index[0].astype(jnp.int32), pad_src])
    src = src.reshape(NW, NCHUNK, K)
    ones16 = jnp.ones((K, 16), jnp.float32)
    zeros16 = jnp.zeros((N_PAD, 16), jnp.float32)
    zeros64 = jnp.zeros((N_PAD, HIDDEN), jnp.float32)

    degp = _deg_kernel(dst, ones16, zeros16)
    xw = _tc_mm(x, W1)
    h1p = _tc1(degp, xw)
    S1 = _agg_kernel(h1p, src, dst, zeros64)
    h2p = _tc2(degp, S1, h1p, b1.reshape(1, HIDDEN), W2)
    S2 = _agg_kernel(h2p, src, dst, zeros64)
    return _tc3(degp, S2, h2p, b2.reshape(1, HIDDEN), Wf, bf.reshape(1, OUT))
